# Initial kernel scaffold; baseline (speedup 1.0000x reference)
import jax
import jax.numpy as jnp
from jax.experimental import pallas as pl


def kernel(coords_batch, W1, b1, W2, b2, Wf, bf):
    raise NotImplementedError("write your pallas kernel here")


# speedup vs baseline: 4.0412x; 4.0412x over previous
import functools

import jax
import jax.numpy as jnp
from jax.experimental import pallas as pl
from jax.experimental.pallas import tpu as pltpu

K = 16
_NEG = None


def _knn_body(n_valid, rows_per_blk, npad, rows_ref, coords_ref, out_ref,
              d2_ref):
    r = pl.program_id(1)
    xs = coords_ref[0, 0, :][None, :]
    ys = coords_ref[0, 1, :][None, :]
    zs = coords_ref[0, 2, :][None, :]
    rows = rows_ref[0, 0]

    sq_c = xs * xs + ys * ys + zs * zs
    sq_r = jnp.sum(rows * rows, axis=1)[:, None]
    dot = jax.lax.dot_general(
        rows, coords_ref[0], (((1,), (0,)), ((), ())),
        preferred_element_type=jnp.float32)
    d2 = sq_r + sq_c - 2.0 * dot

    col = jax.lax.broadcasted_iota(jnp.int32, (rows_per_blk, npad), 1)
    row_g = r * rows_per_blk + jax.lax.broadcasted_iota(
        jnp.int32, (rows_per_blk, npad), 0)
    inf = jnp.float32(jnp.inf)
    d2 = jnp.where((col == row_g) | (col >= n_valid), inf, d2)
    d2_ref[...] = d2

    big = jnp.int32(npad + 1)
    for t in range(K):
        d2 = d2_ref[...]
        m = jnp.min(d2, axis=1)[:, None]
        cand = jnp.where(d2 == m, col, big)
        idx = jnp.min(cand, axis=1)
        out_ref[0, :, t] = idx
        d2_ref[...] = jnp.where(col == idx[:, None], inf, d2)


def _knn_indices(coords_batch, rows_per_blk=256):
    b, n, _ = coords_batch.shape
    npad = ((n + rows_per_blk - 1) // rows_per_blk) * rows_per_blk
    coords_t = jnp.transpose(coords_batch, (0, 2, 1))
    coords_t = jnp.pad(coords_t, ((0, 0), (0, 0), (0, npad - n)))
    nblk = npad // rows_per_blk
    coords_pad = jnp.pad(coords_batch, ((0, 0), (0, npad - n), (0, 0)))
    rows_in = coords_pad.reshape(b, nblk, rows_per_blk, 3)
    out = pl.pallas_call(
        functools.partial(_knn_body, n, rows_per_blk, npad),
        grid=(b, nblk),
        in_specs=[
            pl.BlockSpec((1, 1, rows_per_blk, 3),
                         lambda bi, ri: (bi, ri, 0, 0)),
            pl.BlockSpec((1, 3, npad), lambda bi, ri: (bi, 0, 0)),
        ],
        out_specs=pl.BlockSpec((1, rows_per_blk, K), lambda bi, ri: (bi, ri, 0)),
        out_shape=jax.ShapeDtypeStruct((b, npad, K), jnp.int32),
        scratch_shapes=[pltpu.VMEM((rows_per_blk, npad), jnp.float32)],
    )(rows_in, coords_t)
    return out[:, :n, :]


def _mm1_body(x_ref, w_ref, o_ref):
    o_ref[...] = jax.lax.dot_general(
        x_ref[...], w_ref[...], (((1,), (0,)), ((), ())),
        preferred_element_type=jnp.float32)


def _mm1(x, w, blk=2048):
    m = x.shape[0]
    f = w.shape[1]
    return pl.pallas_call(
        _mm1_body,
        grid=(m // blk,),
        in_specs=[
            pl.BlockSpec((blk, x.shape[1]), lambda i: (i, 0)),
            pl.BlockSpec(w.shape, lambda i: (0, 0)),
        ],
        out_specs=pl.BlockSpec((blk, f), lambda i: (i, 0)),
        out_shape=jax.ShapeDtypeStruct((m, f), jnp.float32),
    )(x, w)


def _mlp_body(t_ref, w2_ref, b2_ref, wf_ref, bf_ref, o_ref):
    h = jax.lax.dot_general(
        t_ref[...], w2_ref[...], (((1,), (0,)), ((), ())),
        preferred_element_type=jnp.float32)
    h = jnp.maximum(h + b2_ref[...][None, :], 0.0)
    o = jax.lax.dot_general(
        h, wf_ref[...], (((1,), (0,)), ((), ())),
        preferred_element_type=jnp.float32)
    o_ref[...] = o + bf_ref[...][None, :]


def _mlp(t, w2s, b2, wf, bf, blk=2048):
    m = t.shape[0]
    f = wf.shape[1]
    return pl.pallas_call(
        _mlp_body,
        grid=(m // blk,),
        in_specs=[
            pl.BlockSpec((blk, t.shape[1]), lambda i: (i, 0)),
            pl.BlockSpec(w2s.shape, lambda i: (0, 0)),
            pl.BlockSpec(b2.shape, lambda i: (0,)),
            pl.BlockSpec(wf.shape, lambda i: (0, 0)),
            pl.BlockSpec(bf.shape, lambda i: (0,)),
        ],
        out_specs=pl.BlockSpec((blk, f), lambda i: (i, 0)),
        out_shape=jax.ShapeDtypeStruct((m, f), jnp.float32),
    )(t, w2s, b2, wf, bf)


def _agg_tmp(x, idx, bias=None, scale=None):
    out = jnp.sum(x[idx], axis=1)
    if scale is not None:
        out = out * scale
    if bias is not None:
        out = jnp.maximum(out + bias[None, :], 0.0)
    return out


def kernel(coords_batch, W1, b1, W2, b2, Wf, bf):
    b, n, _ = coords_batch.shape
    nbr = _knn_indices(coords_batch)
    nbr_g = nbr + (jnp.arange(b, dtype=jnp.int32) * n)[:, None, None]
    nbr_g = nbr_g.reshape(b * n, K)
    self_idx = jnp.arange(b * n, dtype=jnp.int32)[:, None]
    idx = jnp.concatenate([nbr_g, self_idx], axis=1)

    m = b * n
    mpad = ((m + 2047) // 2048) * 2048
    coords_flat = coords_batch.reshape(m, 3)
    coords_flat = jnp.pad(coords_flat, ((0, mpad - m), (0, 0)))
    idx = jnp.pad(idx, ((0, mpad - m), (0, 0)))

    y1 = _mm1(coords_flat, W1)
    inv = jnp.float32(1.0 / (K + 1))
    h1 = _agg_tmp(y1, idx, bias=b1, scale=inv)
    t = _agg_tmp(h1, idx)
    out = _mlp(t, W2 * inv, b2, Wf, bf)
    return out[:m].reshape(b, n, Wf.shape[1])
